# baseline (speedup 1.0000x reference)
import functools

import jax
import jax.numpy as jnp
from jax import lax
from jax.experimental import pallas as pl
from jax.experimental.pallas import tpu as pltpu
from jax.experimental.pallas import tpu_sc as plsc

NUM_FIELDS = 26
VOCAB = 100000
EMBED_DIM = 32
BATCH = 16384
SCALE = NUM_FIELDS ** -0.5

_info = plsc.get_sparse_core_info()
NC, NS, L = _info.num_cores, _info.num_subcores, _info.num_lanes
NW = NC * NS
B_PER_W = BATCH // NW
GCHUNK = 128
NCHUNK = B_PER_W // GCHUNK


def _body(xt_hbm, tables_hbm, out_hbm, idx_v, rows_v, acc_v, sem):
    wid = lax.axis_index("s") * NC + lax.axis_index("c")

    def load_idx(f):
        pltpu.sync_copy(xt_hbm.at[f, wid], idx_v)

    def gather(f, dst_v):
        descs = []
        for c in range(NCHUNK):
            descs.append(pltpu.async_copy(
                tables_hbm.at[f].at[idx_v.at[pl.ds(c * GCHUNK, GCHUNK)]],
                dst_v.at[pl.ds(c * GCHUNK, GCHUNK)],
                sem))
        for d in descs:
            d.wait()

    load_idx(0)
    gather(0, acc_v)

    for f in range(1, NUM_FIELDS - 1):
        load_idx(f)
        gather(f, rows_v)

        @pl.loop(0, B_PER_W)
        def _acc(i):
            for j in range(0, EMBED_DIM, L):
                plsc.addupdate(acc_v.at[i, pl.ds(j, L)],
                               rows_v[i, pl.ds(j, L)])

    load_idx(NUM_FIELDS - 1)
    gather(NUM_FIELDS - 1, rows_v)

    @pl.loop(0, B_PER_W)
    def _fin(i):
        for j in range(0, EMBED_DIM, L):
            s = pl.ds(j, L)
            acc_v[i, s] = (acc_v[i, s] + rows_v[i, s]) * SCALE

    pltpu.sync_copy(acc_v, out_hbm.at[pl.ds(wid * B_PER_W, B_PER_W)])


@jax.jit
def _embed_sum(xt, tables):
    mesh = plsc.VectorSubcoreMesh(core_axis_name="c", subcore_axis_name="s")
    return pl.kernel(
        _body,
        out_type=jax.ShapeDtypeStruct((BATCH, EMBED_DIM), jnp.float32),
        mesh=mesh,
        scratch_types=[
            pltpu.VMEM((B_PER_W,), jnp.int32),
            pltpu.VMEM((B_PER_W, EMBED_DIM), jnp.float32),
            pltpu.VMEM((B_PER_W, EMBED_DIM), jnp.float32),
            pltpu.SemaphoreType.DMA,
        ],
        compiler_params=pltpu.CompilerParams(use_tc_tiling_on_sc=False),
    )(xt, tables)


def kernel(x, tables):
    if x.ndim == 1:
        x = x[:, None]
    xt = x.T.reshape(NUM_FIELDS, NW, B_PER_W)
    return _embed_sum(xt, tables)


# speedup vs baseline: 1.0463x; 1.0463x over previous
import functools

import jax
import jax.numpy as jnp
from jax import lax
from jax.experimental import pallas as pl
from jax.experimental.pallas import tpu as pltpu
from jax.experimental.pallas import tpu_sc as plsc

NUM_FIELDS = 26
VOCAB = 100000
EMBED_DIM = 32
BATCH = 16384
SCALE = NUM_FIELDS ** -0.5

_info = plsc.get_sparse_core_info()
NC, NS, L = _info.num_cores, _info.num_subcores, _info.num_lanes
NW = NC * NS
B_PER_W = BATCH // NW
GCHUNK = 128
NCHUNK = B_PER_W // GCHUNK
RUNROLL = 8


def _body(xt_hbm, tables_hbm, out_hbm, idx_v, rows_v, acc_v, sem0, sem1):
    wid = lax.axis_index("s") * NC + lax.axis_index("c")
    sems = (sem0, sem1)

    pltpu.sync_copy(xt_hbm.at[wid], idx_v)

    def fire(f, dst_v, sem):
        return [
            pltpu.async_copy(
                tables_hbm.at[f].at[idx_v.at[f, pl.ds(c * GCHUNK, GCHUNK)]],
                dst_v.at[pl.ds(c * GCHUNK, GCHUNK)],
                sem)
            for c in range(NCHUNK)
        ]

    def drain(descs):
        for d in descs:
            d.wait()

    d_acc = fire(0, acc_v, sems[0])
    pending = fire(1, rows_v.at[1], sems[1])
    drain(d_acc)

    for f in range(1, NUM_FIELDS):
        buf = f % 2
        if f + 1 < NUM_FIELDS:
            nxt = fire(f + 1, rows_v.at[(f + 1) % 2], sems[(f + 1) % 2])
        drain(pending)
        if f + 1 < NUM_FIELDS:
            pending = nxt

        if f < NUM_FIELDS - 1:
            @pl.loop(0, B_PER_W, step=RUNROLL)
            def _acc(i):
                for r in range(RUNROLL):
                    for j in range(0, EMBED_DIM, L):
                        plsc.addupdate(acc_v.at[i + r, pl.ds(j, L)],
                                       rows_v[buf, i + r, pl.ds(j, L)])
        else:
            @pl.loop(0, B_PER_W, step=RUNROLL)
            def _fin(i):
                for r in range(RUNROLL):
                    for j in range(0, EMBED_DIM, L):
                        s = pl.ds(j, L)
                        acc_v[i + r, s] = (
                            acc_v[i + r, s] + rows_v[buf, i + r, s]) * SCALE

    pltpu.sync_copy(acc_v, out_hbm.at[pl.ds(wid * B_PER_W, B_PER_W)])


@jax.jit
def _embed_sum(xt, tables):
    mesh = plsc.VectorSubcoreMesh(core_axis_name="c", subcore_axis_name="s")
    return pl.kernel(
        _body,
        out_type=jax.ShapeDtypeStruct((BATCH, EMBED_DIM), jnp.float32),
        mesh=mesh,
        scratch_types=[
            pltpu.VMEM((NUM_FIELDS, B_PER_W), jnp.int32),
            pltpu.VMEM((2, B_PER_W, EMBED_DIM), jnp.float32),
            pltpu.VMEM((B_PER_W, EMBED_DIM), jnp.float32),
            pltpu.SemaphoreType.DMA,
            pltpu.SemaphoreType.DMA,
        ],
        compiler_params=pltpu.CompilerParams(use_tc_tiling_on_sc=False),
    )(xt, tables)


def kernel(x, tables):
    if x.ndim == 1:
        x = x[:, None]
    xt = x.T.reshape(NUM_FIELDS, NW, B_PER_W).transpose(1, 0, 2)
    return _embed_sum(xt, tables)


# speedup vs baseline: 3.4020x; 3.2513x over previous
import jax
import jax.numpy as jnp
from jax import lax
from jax.experimental import pallas as pl
from jax.experimental.pallas import tpu as pltpu
from jax.experimental.pallas import tpu_sc as plsc

NUM_FIELDS = 26
VOCAB = 100000
EMBED_DIM = 32
BATCH = 16384
SCALE = NUM_FIELDS ** -0.5

_info = plsc.get_sparse_core_info()
NC, NS, L = _info.num_cores, _info.num_subcores, _info.num_lanes
NW = NC * NS
ICHUNK = 8192
NICHUNK = BATCH // ICHUNK
UNROLL = 4


def _body(xt_hbm, tabT_hbm, outT_hbm, idx_v, row_v, acc_v):
    wid = lax.axis_index("s") * NC + lax.axis_index("c")

    for f in range(NUM_FIELDS):
        pltpu.sync_copy(tabT_hbm.at[f, wid], row_v)
        for c in range(NICHUNK):
            pltpu.sync_copy(xt_hbm.at[f, pl.ds(c * ICHUNK, ICHUNK)], idx_v)
            cbase = c * ICHUNK

            @pl.loop(0, ICHUNK, step=L * UNROLL)
            def _gather(i):
                for u in range(UNROLL):
                    sl = pl.ds(i + u * L, L)
                    vals = plsc.load_gather(row_v, [idx_v[sl]])
                    asl = pl.ds(cbase + i + u * L, L)
                    if f == 0:
                        acc_v[asl] = vals
                    else:
                        plsc.addupdate(acc_v.at[asl], vals)

    @pl.loop(0, BATCH, step=L * UNROLL)
    def _scale(i):
        for u in range(UNROLL):
            sl = pl.ds(i + u * L, L)
            acc_v[sl] = acc_v[sl] * SCALE

    pltpu.sync_copy(acc_v, outT_hbm.at[wid])


def _embed_sum(xt, tabT):
    mesh = plsc.VectorSubcoreMesh(core_axis_name="c", subcore_axis_name="s")
    return pl.kernel(
        _body,
        out_type=jax.ShapeDtypeStruct((EMBED_DIM, BATCH), jnp.float32),
        mesh=mesh,
        scratch_types=[
            pltpu.VMEM((ICHUNK,), jnp.int32),
            pltpu.VMEM((VOCAB,), jnp.float32),
            pltpu.VMEM((BATCH,), jnp.float32),
        ],
        compiler_params=pltpu.CompilerParams(needs_layout_passes=False),
    )(xt, tabT)


def kernel(x, tables):
    if x.ndim == 1:
        x = x[:, None]
    xt = x.T
    tabT = tables.transpose(0, 2, 1)
    outT = _embed_sum(xt, tabT)
    return outT.T


# speedup vs baseline: 4.9958x; 1.4685x over previous
import jax
import jax.numpy as jnp
from jax import lax
from jax.experimental import pallas as pl
from jax.experimental.pallas import tpu as pltpu
from jax.experimental.pallas import tpu_sc as plsc

NUM_FIELDS = 26
VOCAB = 100000
EMBED_DIM = 32
BATCH = 16384
SCALE = NUM_FIELDS ** -0.5

_info = plsc.get_sparse_core_info()
NC, NS, L = _info.num_cores, _info.num_subcores, _info.num_lanes
NW = NC * NS
ICHUNK = 8192
NICHUNK = BATCH // ICHUNK
UNROLL = 4


def _body(xt_hbm, tabT_hbm, outT_hbm, idx_v, row_v, acc_v):
    wid = lax.axis_index("s") * NC + lax.axis_index("c")

    for f in range(NUM_FIELDS):
        pltpu.sync_copy(tabT_hbm.at[f, wid], row_v)
        for c in range(NICHUNK):
            pltpu.sync_copy(xt_hbm.at[f, pl.ds(c * ICHUNK, ICHUNK)], idx_v)
            cbase = c * ICHUNK

            @plsc.parallel_loop(0, ICHUNK, step=L, unroll=UNROLL)
            def _gather(i):
                sl = pl.ds(i, L)
                vals = plsc.load_gather(row_v, [idx_v[sl]])
                asl = pl.ds(cbase + i, L)
                if f == 0:
                    acc_v[asl] = vals
                else:
                    plsc.addupdate(acc_v.at[asl], vals)

    @plsc.parallel_loop(0, BATCH, step=L, unroll=UNROLL)
    def _scale(i):
        sl = pl.ds(i, L)
        acc_v[sl] = acc_v[sl] * SCALE

    pltpu.sync_copy(acc_v, outT_hbm.at[wid])


def _embed_sum(xt, tabT):
    mesh = plsc.VectorSubcoreMesh(core_axis_name="c", subcore_axis_name="s")
    return pl.kernel(
        _body,
        out_type=jax.ShapeDtypeStruct((EMBED_DIM, BATCH), jnp.float32),
        mesh=mesh,
        scratch_types=[
            pltpu.VMEM((ICHUNK,), jnp.int32),
            pltpu.VMEM((VOCAB,), jnp.float32),
            pltpu.VMEM((BATCH,), jnp.float32),
        ],
        compiler_params=pltpu.CompilerParams(needs_layout_passes=False),
    )(xt, tabT)


def kernel(x, tables):
    if x.ndim == 1:
        x = x[:, None]
    xt = x.T
    tabT = tables.transpose(0, 2, 1)
    outT = _embed_sum(xt, tabT)
    return outT.T


# speedup vs baseline: 5.0478x; 1.0104x over previous
import jax
import jax.numpy as jnp
from jax import lax
from jax.experimental import pallas as pl
from jax.experimental.pallas import tpu as pltpu
from jax.experimental.pallas import tpu_sc as plsc

NUM_FIELDS = 26
VOCAB = 100000
EMBED_DIM = 32
BATCH = 16384
SCALE = NUM_FIELDS ** -0.5

_info = plsc.get_sparse_core_info()
NC, NS, L = _info.num_cores, _info.num_subcores, _info.num_lanes
NW = NC * NS
ICHUNK = 8192
NICHUNK = BATCH // ICHUNK
UNROLL = 8


def _body(xt_hbm, tabT_hbm, outT_hbm, idx_v, row_v, acc_v):
    wid = lax.axis_index("s") * NC + lax.axis_index("c")

    for f in range(NUM_FIELDS):
        pltpu.sync_copy(tabT_hbm.at[f, wid], row_v)
        for c in range(NICHUNK):
            pltpu.sync_copy(xt_hbm.at[f, pl.ds(c * ICHUNK, ICHUNK)], idx_v)
            cbase = c * ICHUNK

            @plsc.parallel_loop(0, ICHUNK, step=L, unroll=UNROLL)
            def _gather(i):
                sl = pl.ds(i, L)
                vals = plsc.load_gather(row_v, [idx_v[sl]])
                asl = pl.ds(cbase + i, L)
                if f == 0:
                    acc_v[asl] = vals
                else:
                    plsc.addupdate(acc_v.at[asl], vals)

    @plsc.parallel_loop(0, BATCH, step=L, unroll=UNROLL)
    def _scale(i):
        sl = pl.ds(i, L)
        acc_v[sl] = acc_v[sl] * SCALE

    pltpu.sync_copy(acc_v, outT_hbm.at[wid])


def _embed_sum(xt, tabT):
    mesh = plsc.VectorSubcoreMesh(core_axis_name="c", subcore_axis_name="s")
    return pl.kernel(
        _body,
        out_type=jax.ShapeDtypeStruct((EMBED_DIM, BATCH), jnp.float32),
        mesh=mesh,
        scratch_types=[
            pltpu.VMEM((ICHUNK,), jnp.int32),
            pltpu.VMEM((VOCAB,), jnp.float32),
            pltpu.VMEM((BATCH,), jnp.float32),
        ],
        compiler_params=pltpu.CompilerParams(needs_layout_passes=False),
    )(xt, tabT)


def kernel(x, tables):
    if x.ndim == 1:
        x = x[:, None]
    xt = x.T
    tabT = tables.transpose(0, 2, 1)
    outT = _embed_sum(xt, tabT)
    return outT.T
